# baseline (speedup 1.0000x reference)
import jax
import jax.numpy as jnp
import numpy as np
from jax import lax
from jax.experimental import pallas as pl
from jax.experimental.pallas import tpu as pltpu
from jax.experimental.pallas import tpu_sc as plsc

_PAT = np.array(sorted([1, 4, 8, 11, 15, 19, 22, 26, 30, 33, 37, 41, 44,
                        48, 52, 55, 59, 63, 66, 70, 74, 77, 81, 85, 88, 92]),
                dtype=np.int32)

_B = 1024
_T = 50
_H = 16
_IN_W = 100
_OUT_W = 26
_NC = 2
_NS = 16
_NW = _NC * _NS
_P = _T * _OUT_W
_K = (_P + _NW - 1) // _NW
_NSEM = 8
_TBL = 1344


def _body(in_hbm4, tbl_hbm, out_hbm4, tbl_v, *sems):
    tin = in_hbm4.reshape(_T * _IN_W, _H, _B)
    tout = out_hbm4.reshape(_P, _H, _B)

    cid = lax.axis_index("c")
    sid = lax.axis_index("s")
    wid = sid * _NC + cid

    pltpu.sync_copy(tbl_hbm, tbl_v)

    def src_of(k):
        v = tbl_v[pl.ds(wid + k * _NW, 16)]
        return v[0]

    def pair_of(k):
        return wid + k * _NW

    def start(k, s):
        pltpu.async_copy(tin.at[src_of(k)], tout.at[pair_of(k)], sems[s])

    def wait(k, s):
        pltpu.make_async_copy(
            tin.at[src_of(k)], tout.at[pair_of(k)], sems[s]).wait()

    for k0 in range(_NSEM):
        @pl.when(pair_of(k0) < _P)
        def _prime():
            start(k0, k0 % _NSEM)

    _NG = -(-_K // _NSEM)

    @pl.loop(0, _NG)
    def group(g):
        for i in range(_NSEM):
            k = g * _NSEM + i

            @pl.when(pair_of(k) < _P)
            def _do():
                wait(k, i)

                @pl.when(pair_of(k + _NSEM) < _P)
                def _ahead():
                    start(k + _NSEM, i)


@jax.jit
def _run(tin4, tbl):
    fn = pl.kernel(
        _body,
        out_type=jax.ShapeDtypeStruct((_T, _OUT_W, _H, _B), jnp.float32),
        mesh=plsc.VectorSubcoreMesh(core_axis_name="c", subcore_axis_name="s"),
        scratch_types=(
            [pltpu.VMEM((_TBL,), jnp.int32)]
            + [pltpu.SemaphoreType.DMA for _ in range(_NSEM)]
        ),
        compiler_params=pltpu.CompilerParams(needs_layout_passes=False),
    )
    return fn(tin4, tbl)


def kernel(inputs):
    p = np.arange(_TBL, dtype=np.int64)
    pc = np.minimum(p, _P - 1)
    tbl = ((pc // _OUT_W) * _IN_W + _PAT[pc % _OUT_W]).astype(np.int32)
    tin = jnp.transpose(inputs, (1, 3, 2, 0))
    tout = _run(tin, jnp.asarray(tbl))
    return jnp.transpose(tout, (3, 0, 2, 1))


# speedup vs baseline: 32.8452x; 32.8452x over previous
import jax
import jax.numpy as jnp
import numpy as np
from jax import lax
from jax.experimental import pallas as pl
from jax.experimental.pallas import tpu as pltpu
from jax.experimental.pallas import tpu_sc as plsc

_PAT = np.array(sorted([1, 4, 8, 11, 15, 19, 22, 26, 30, 33, 37, 41, 44,
                        48, 52, 55, 59, 63, 66, 70, 74, 77, 81, 85, 88, 92]),
                dtype=np.int32)

_B = 1024
_T = 50
_H = 16
_IN_W = 100
_OUT_W = 26
_NC = 2
_NS = 16
_NW = _NC * _NS
_P = _T * _OUT_W
_K = (_P + _NW - 1) // _NW
_NBUF = 7
_LOOK = 3
_TBL = 1344


def _body(in_hbm4, tbl_hbm, out_hbm4,
          buf0, buf1, buf2, buf3, buf4, buf5, buf6, tbl_v,
          si0, si1, si2, si3, si4, si5, si6,
          so0, so1, so2, so3, so4, so5, so6):
    tin = in_hbm4.reshape(_T * _IN_W, _H, _B)
    tout = out_hbm4.reshape(_P, _H, _B)
    bufs = [buf0, buf1, buf2, buf3, buf4, buf5, buf6]
    sins = [si0, si1, si2, si3, si4, si5, si6]
    souts = [so0, so1, so2, so3, so4, so5, so6]

    cid = lax.axis_index("c")
    sid = lax.axis_index("s")
    wid = sid * _NC + cid

    pltpu.sync_copy(tbl_hbm, tbl_v)

    def src_of(k):
        v = tbl_v[pl.ds(wid + k * _NW, 16)]
        return v[0]

    def pair_of(k):
        return wid + k * _NW

    def start_in(k, b):
        pltpu.async_copy(tin.at[src_of(k)], bufs[b], sins[b])

    def wait_in(k, b):
        pltpu.make_async_copy(tin.at[src_of(k)], bufs[b], sins[b]).wait()

    def start_out(k, b):
        pltpu.async_copy(bufs[b], tout.at[pair_of(k)], souts[b])

    def wait_out(k, b):
        pltpu.make_async_copy(bufs[b], tout.at[pair_of(k)], souts[b]).wait()

    for k0 in range(_LOOK):
        @pl.when(pair_of(k0) < _P)
        def _prime():
            start_in(k0, k0 % _NBUF)

    _NG = -(-_K // _NBUF)

    @pl.loop(0, _NG)
    def group(g):
        for i in range(_NBUF):
            k = g * _NBUF + i

            @pl.when(pair_of(k) < _P)
            def _do():
                wait_in(k, i)
                start_out(k, i)

                @pl.when(pair_of(k + _LOOK) < _P)
                def _ahead():
                    start_in(k + _LOOK, (i + _LOOK) % _NBUF)

            @pl.when((k >= _LOOK) & (pair_of(k - _LOOK) < _P))
            def _drain():
                wait_out(k - _LOOK, (i - _LOOK) % _NBUF)

    for kk in range(_NG * _NBUF - _LOOK, _K):
        @pl.when(pair_of(kk) < _P)
        def _tail():
            wait_out(kk, kk % _NBUF)


@jax.jit
def _run(tin4, tbl):
    fn = pl.kernel(
        _body,
        out_type=jax.ShapeDtypeStruct((_T, _OUT_W, _H, _B), jnp.float32),
        mesh=plsc.VectorSubcoreMesh(core_axis_name="c", subcore_axis_name="s"),
        scratch_types=(
            [pltpu.VMEM((_H, _B), jnp.float32) for _ in range(_NBUF)]
            + [pltpu.VMEM((_TBL,), jnp.int32)]
            + [pltpu.SemaphoreType.DMA for _ in range(2 * _NBUF)]
        ),
        compiler_params=pltpu.CompilerParams(needs_layout_passes=False),
    )
    return fn(tin4, tbl)


def kernel(inputs):
    p = np.arange(_TBL, dtype=np.int64)
    pc = np.minimum(p, _P - 1)
    tbl = ((pc // _OUT_W) * _IN_W + _PAT[pc % _OUT_W]).astype(np.int32)
    tin = jnp.transpose(inputs, (1, 3, 2, 0))
    tout = _run(tin, jnp.asarray(tbl))
    return jnp.transpose(tout, (3, 0, 2, 1))


# speedup vs baseline: 33.2086x; 1.0111x over previous
import jax
import jax.numpy as jnp
import numpy as np
from jax import lax
from jax.experimental import pallas as pl
from jax.experimental.pallas import tpu as pltpu
from jax.experimental.pallas import tpu_sc as plsc

_PAT = np.array(sorted([1, 4, 8, 11, 15, 19, 22, 26, 30, 33, 37, 41, 44,
                        48, 52, 55, 59, 63, 66, 70, 74, 77, 81, 85, 88, 92]),
                dtype=np.int32)

_B = 1024
_T = 50
_H = 16
_IN_W = 100
_OUT_W = 26
_NC = 2
_NS = 16
_NW = _NC * _NS
_P = _T * _OUT_W
_K = (_P + _NW - 1) // _NW
_NBUF = 7
_LOOK = 3
_TBL = 48


def _body(in_hbm4, out_hbm4,
          buf0, buf1, buf2, buf3, buf4, buf5, buf6, tbl_v,
          si0, si1, si2, si3, si4, si5, si6,
          so0, so1, so2, so3, so4, so5, so6):
    tin = in_hbm4.reshape(_T * _IN_W, _H, _B)
    tout = out_hbm4.reshape(_P, _H, _B)
    bufs = [buf0, buf1, buf2, buf3, buf4, buf5, buf6]
    sins = [si0, si1, si2, si3, si4, si5, si6]
    souts = [so0, so1, so2, so3, so4, so5, so6]

    cid = lax.axis_index("c")
    sid = lax.axis_index("s")
    wid = sid * _NC + cid

    for jj in range(_OUT_W):
        tbl_v[jj] = int(_PAT[jj])

    def src_of(k):
        p = wid + k * _NW
        t = p // _OUT_W
        j = p - t * _OUT_W
        return t * _IN_W + tbl_v[j]

    def pair_of(k):
        return wid + k * _NW

    def start_in(k, b):
        pltpu.async_copy(tin.at[src_of(k)], bufs[b], sins[b])

    def wait_in(k, b):
        pltpu.make_async_copy(tin.at[src_of(k)], bufs[b], sins[b]).wait()

    def start_out(k, b):
        pltpu.async_copy(bufs[b], tout.at[pair_of(k)], souts[b])

    def wait_out(k, b):
        pltpu.make_async_copy(bufs[b], tout.at[pair_of(k)], souts[b]).wait()

    for k0 in range(_LOOK):
        @pl.when(pair_of(k0) < _P)
        def _prime():
            start_in(k0, k0 % _NBUF)

    _NG = -(-_K // _NBUF)

    @pl.loop(0, _NG)
    def group(g):
        for i in range(_NBUF):
            k = g * _NBUF + i

            @pl.when(pair_of(k) < _P)
            def _do():
                wait_in(k, i)
                start_out(k, i)

                @pl.when(pair_of(k + _LOOK) < _P)
                def _ahead():
                    start_in(k + _LOOK, (i + _LOOK) % _NBUF)

            @pl.when((k >= _LOOK) & (pair_of(k - _LOOK) < _P))
            def _drain():
                wait_out(k - _LOOK, (i - _LOOK) % _NBUF)

    for kk in range(_NG * _NBUF - _LOOK, _K):
        @pl.when(pair_of(kk) < _P)
        def _tail():
            wait_out(kk, kk % _NBUF)


@jax.jit
def _run(tin4):
    fn = pl.kernel(
        _body,
        out_type=jax.ShapeDtypeStruct((_T, _OUT_W, _H, _B), jnp.float32),
        mesh=plsc.VectorSubcoreMesh(core_axis_name="c", subcore_axis_name="s"),
        scratch_types=(
            [pltpu.VMEM((_H, _B), jnp.float32) for _ in range(_NBUF)]
            + [pltpu.SMEM((_TBL,), jnp.int32)]
            + [pltpu.SemaphoreType.DMA for _ in range(2 * _NBUF)]
        ),
        compiler_params=pltpu.CompilerParams(needs_layout_passes=False),
    )
    return fn(tin4)


def kernel(inputs):
    tin = jnp.transpose(inputs, (1, 3, 2, 0))
    tout = _run(tin)
    return jnp.transpose(tout, (3, 0, 2, 1))
